# Initial kernel scaffold; baseline (speedup 1.0000x reference)
import jax
import jax.numpy as jnp
from jax.experimental import pallas as pl


def kernel(x, edge_index):
    raise NotImplementedError("write your pallas kernel here")


# speedup vs baseline: 2.6756x; 2.6756x over previous
import functools

import jax
import jax.numpy as jnp
from jax import lax
from jax.experimental import pallas as pl
from jax.experimental.pallas import tpu as pltpu
from jax.experimental.pallas import tpu_sc as plsc

N = 10000
NP = 10240
NR = NP // 128
E = 320000
D = 128

NWORK = 32
EW = E // NWORK


def _noise_pad():
    v = jax.random.uniform(jax.random.key(1), (1, N), dtype=jnp.float32) * 0.1
    return jnp.pad(v, ((0, 0), (0, NP - N)), constant_values=jnp.inf)


def _sc_hist_body(src_hbm, out_hbm, idx_v, hist_v):
    c = lax.axis_index("c")
    s = lax.axis_index("s")
    w = s * 2 + c
    base = w * EW
    pltpu.sync_copy(src_hbm.at[pl.ds(base, EW)], idx_v)
    zeros16 = jnp.zeros((16,), jnp.float32)

    def zbody(i, carry):
        hist_v[i >> 3, pl.ds((i & 7) * 16, 16)] = zeros16
        return carry

    lax.fori_loop(0, NP // 16, zbody, 0)
    ones16 = jnp.ones((16,), jnp.float32)

    def body(i, carry):
        iv = idx_v[pl.ds(i * 16, 16)]
        plsc.addupdate_scatter(hist_v, [iv >> 7, iv & 127], ones16)
        return carry

    lax.fori_loop(0, EW // 16, body, 0)
    pltpu.sync_copy(hist_v, out_hbm.at[w])


_sc_hist = functools.partial(
    pl.kernel,
    out_type=jax.ShapeDtypeStruct((NWORK, NR, 128), jnp.float32),
    mesh=plsc.VectorSubcoreMesh(core_axis_name="c", subcore_axis_name="s",
                                num_cores=2, num_subcores=16),
    compiler_params=pltpu.CompilerParams(needs_layout_passes=False),
    scratch_types=[
        pltpu.VMEM((EW,), jnp.int32),
        pltpu.VMEM((NR, 128), jnp.float32),
    ],
)(_sc_hist_body)


def _tc_keys_body(part_ref, noise_ref, keys_ref):
    keys_ref[...] = noise_ref[...] + jnp.sum(part_ref[...], axis=0)


def _tc_keys(partials, noise):
    return pl.pallas_call(
        _tc_keys_body,
        out_shape=jax.ShapeDtypeStruct((NR, 128), jnp.float32),
    )(partials, noise)


BI = 256


def _tc_rank_body(kcol_ref, krow_ref, rank_ref):
    g = pl.program_id(0)
    ki = kcol_ref[...]
    kj = krow_ref[...]
    irow = g * BI + lax.broadcasted_iota(jnp.int32, (BI, 1), 0)
    jcol = lax.broadcasted_iota(jnp.int32, (1, NP), 1)
    lt = kj < ki
    tie = (kj == ki) & (jcol < irow)
    cnt = jnp.sum((lt | tie).astype(jnp.int32), axis=1, keepdims=True)
    rank_ref[...] = cnt


def _tc_rank(kcol, krow):
    return pl.pallas_call(
        _tc_rank_body,
        grid=(NP // BI,),
        in_specs=[
            pl.BlockSpec((BI, 1), lambda g: (g, 0)),
            pl.BlockSpec((1, NP), lambda g: (0, 0)),
        ],
        out_specs=pl.BlockSpec((BI, 1), lambda g: (g, 0)),
        out_shape=jax.ShapeDtypeStruct((NP, 1), jnp.int32),
    )(kcol, krow)


QW = NP // 16


def _sc_sg_body(rank_hbm, x_hbm, sidx_hbm, out_hbm,
                rank_v, vals_v, sidx_v, gidx_v, rows_v, spmem_sidx, sem):
    s = lax.axis_index("s")
    base = s * QW
    pltpu.sync_copy(rank_hbm.at[pl.ds(base, QW)], rank_v)
    iota16 = lax.iota(jnp.int32, 16)

    def vbody(k, carry):
        vals_v[pl.ds(k * 16, 16)] = base + k * 16 + iota16
        return carry

    lax.fori_loop(0, QW // 16, vbody, 0)

    def gbody(g, carry):
        cps = []
        for t in range(8):
            j = g * 8 + t
            idx16 = rank_v[pl.ds(j * 16, 16)]
            cps.append(pltpu.async_copy(vals_v.at[pl.ds(j * 16, 16)],
                                        spmem_sidx.at[idx16], sem))
        for cp in cps:
            cp.wait()
        return carry

    lax.fori_loop(0, QW // 128, gbody, 0)
    plsc.subcore_barrier()
    pltpu.sync_copy(spmem_sidx.at[pl.ds(base, QW)], sidx_v)
    pltpu.sync_copy(sidx_v, sidx_hbm.at[pl.ds(base, QW)])
    nmax = jnp.full((16,), N - 1, jnp.int32)

    def cl_body(k, carry):
        gidx_v[k >> 3, pl.ds((k & 7) * 16, 16)] = jnp.minimum(
            sidx_v[pl.ds(k * 16, 16)], nmax)
        return carry

    lax.fori_loop(0, QW // 16, cl_body, 0)
    gcps = [pltpu.async_copy(x_hbm.at[gidx_v.at[j]],
                             rows_v.at[pl.ds(j * 128, 128)], sem)
            for j in range(QW // 128)]
    for cp in gcps:
        cp.wait()

    @pl.when(s < 15)
    def _():
        pltpu.sync_copy(rows_v, out_hbm.at[pl.ds(base, QW)])

    @pl.when(s == 15)
    def _():
        pltpu.sync_copy(rows_v.at[pl.ds(0, N - 15 * QW)],
                        out_hbm.at[pl.ds(base, N - 15 * QW)])


_sc_sg = functools.partial(
    pl.kernel,
    out_type=(jax.ShapeDtypeStruct((NP,), jnp.int32),
              jax.ShapeDtypeStruct((N, D), jnp.float32)),
    mesh=plsc.VectorSubcoreMesh(core_axis_name="c", subcore_axis_name="s",
                                num_cores=1, num_subcores=16),
    compiler_params=pltpu.CompilerParams(needs_layout_passes=False),
    scratch_types=[
        pltpu.VMEM((QW,), jnp.int32),
        pltpu.VMEM((QW,), jnp.int32),
        pltpu.VMEM((QW,), jnp.int32),
        pltpu.VMEM((QW // 128, 128), jnp.int32),
        pltpu.VMEM((QW, D), jnp.float32),
        pltpu.VMEM_SHARED((NP,), jnp.int32),
        pltpu.SemaphoreType.DMA,
    ],
)(_sc_sg_body)


def kernel(x, edge_index):
    src = edge_index[0]
    partials = _sc_hist(src)
    keys = _tc_keys(partials, _noise_pad().reshape(NR, 128))
    rank = _tc_rank(keys.reshape(NP, 1), keys.reshape(1, NP))
    sidx_pad, x_sorted = _sc_sg(rank.reshape(NP), x)
    return (x_sorted[None], sidx_pad[:N][None])


# speedup vs baseline: 3.2018x; 1.1966x over previous
import functools

import jax
import jax.numpy as jnp
from jax import lax
from jax.experimental import pallas as pl
from jax.experimental.pallas import tpu as pltpu
from jax.experimental.pallas import tpu_sc as plsc

N = 10000
NP = 10240
NR = NP // 128
E = 320000
D = 128

NWORK = 32
EW = E // NWORK


def _noise_pad():
    v = jax.random.uniform(jax.random.key(1), (1, N), dtype=jnp.float32) * 0.1
    return jnp.pad(v, ((0, 0), (0, NP - N)), constant_values=jnp.inf)


def _sc_hist_body(src_hbm, out_hbm, idx_v, hist_v):
    c = lax.axis_index("c")
    s = lax.axis_index("s")
    w = s * 2 + c
    base = w * EW
    pltpu.sync_copy(src_hbm.at[pl.ds(base, EW)], idx_v)
    zeros16 = jnp.zeros((16,), jnp.float32)

    def zbody(i, carry):
        hist_v[i >> 3, pl.ds((i & 7) * 16, 16)] = zeros16
        return carry

    lax.fori_loop(0, NP // 16, zbody, 0)
    ones16 = jnp.ones((16,), jnp.float32)

    def body(i, carry):
        iv = idx_v[pl.ds(i * 16, 16)]
        plsc.addupdate_scatter(hist_v, [iv >> 7, iv & 127], ones16)
        return carry

    lax.fori_loop(0, EW // 16, body, 0)
    pltpu.sync_copy(hist_v, out_hbm.at[w])


_sc_hist = functools.partial(
    pl.kernel,
    out_type=jax.ShapeDtypeStruct((NWORK, NR, 128), jnp.float32),
    mesh=plsc.VectorSubcoreMesh(core_axis_name="c", subcore_axis_name="s",
                                num_cores=2, num_subcores=16),
    compiler_params=pltpu.CompilerParams(needs_layout_passes=False),
    scratch_types=[
        pltpu.VMEM((EW,), jnp.int32),
        pltpu.VMEM((NR, 128), jnp.float32),
    ],
)(_sc_hist_body)


def _tc_keys_body(part_ref, noise_ref, keys_ref):
    keys_ref[...] = noise_ref[...] + jnp.sum(part_ref[...], axis=0)


def _tc_keys(partials, noise):
    return pl.pallas_call(
        _tc_keys_body,
        out_shape=jax.ShapeDtypeStruct((NR, 128), jnp.float32),
    )(partials, noise)


BI = 1024
BJ = 1024


def _tc_rank_body(kcol_ref, krow_ref, rank_ref):
    gi = pl.program_id(0)
    gj = pl.program_id(1)
    ki = kcol_ref[...]
    kj = krow_ref[...]
    i_start = gi * BI
    j_start = gj * BJ

    @pl.when(gj == 0)
    def _():
        rank_ref[...] = jnp.zeros((BI, 1), jnp.int32)

    @pl.when(j_start + BJ <= i_start)
    def _():
        rank_ref[...] += jnp.sum((kj <= ki).astype(jnp.int32), axis=1,
                                 keepdims=True)

    @pl.when(j_start >= i_start + BI)
    def _():
        rank_ref[...] += jnp.sum((kj < ki).astype(jnp.int32), axis=1,
                                 keepdims=True)

    @pl.when((j_start + BJ > i_start) & (j_start < i_start + BI))
    def _():
        irow = i_start + lax.broadcasted_iota(jnp.int32, (BI, 1), 0)
        jcol = j_start + lax.broadcasted_iota(jnp.int32, (1, BJ), 1)
        lt = kj < ki
        tie = (kj == ki) & (jcol < irow)
        rank_ref[...] += jnp.sum((lt | tie).astype(jnp.int32), axis=1,
                                 keepdims=True)


def _tc_rank(kcol, krow):
    return pl.pallas_call(
        _tc_rank_body,
        grid=(NP // BI, NP // BJ),
        in_specs=[
            pl.BlockSpec((BI, 1), lambda gi, gj: (gi, 0)),
            pl.BlockSpec((1, BJ), lambda gi, gj: (0, gj)),
        ],
        out_specs=pl.BlockSpec((BI, 1), lambda gi, gj: (gi, 0)),
        out_shape=jax.ShapeDtypeStruct((NP, 1), jnp.int32),
    )(kcol, krow)


QW = NP // 16


def _sc_sg_body(rank_hbm, x_hbm, sidx_hbm, out_hbm,
                rank_v, vals_v, sidx_v, gidx_v, rows_v, spmem_sidx, sem):
    s = lax.axis_index("s")
    base = s * QW
    pltpu.sync_copy(rank_hbm.at[pl.ds(base, QW)], rank_v)
    iota16 = lax.iota(jnp.int32, 16)

    def vbody(k, carry):
        vals_v[pl.ds(k * 16, 16)] = base + k * 16 + iota16
        return carry

    lax.fori_loop(0, QW // 16, vbody, 0)

    def gbody(g, carry):
        cps = []
        for t in range(8):
            j = g * 8 + t
            idx16 = rank_v[pl.ds(j * 16, 16)]
            cps.append(pltpu.async_copy(vals_v.at[pl.ds(j * 16, 16)],
                                        spmem_sidx.at[idx16], sem))
        for cp in cps:
            cp.wait()
        return carry

    lax.fori_loop(0, QW // 128, gbody, 0)
    plsc.subcore_barrier()
    pltpu.sync_copy(spmem_sidx.at[pl.ds(base, QW)], sidx_v)
    pltpu.sync_copy(sidx_v, sidx_hbm.at[pl.ds(base, QW)])
    nmax = jnp.full((16,), N - 1, jnp.int32)

    def cl_body(k, carry):
        gidx_v[k >> 3, pl.ds((k & 7) * 16, 16)] = jnp.minimum(
            sidx_v[pl.ds(k * 16, 16)], nmax)
        return carry

    lax.fori_loop(0, QW // 16, cl_body, 0)
    gcps = [pltpu.async_copy(x_hbm.at[gidx_v.at[j]],
                             rows_v.at[pl.ds(j * 128, 128)], sem)
            for j in range(QW // 128)]
    for cp in gcps:
        cp.wait()

    @pl.when(s < 15)
    def _():
        pltpu.sync_copy(rows_v, out_hbm.at[pl.ds(base, QW)])

    @pl.when(s == 15)
    def _():
        pltpu.sync_copy(rows_v.at[pl.ds(0, N - 15 * QW)],
                        out_hbm.at[pl.ds(base, N - 15 * QW)])


_sc_sg = functools.partial(
    pl.kernel,
    out_type=(jax.ShapeDtypeStruct((NP,), jnp.int32),
              jax.ShapeDtypeStruct((N, D), jnp.float32)),
    mesh=plsc.VectorSubcoreMesh(core_axis_name="c", subcore_axis_name="s",
                                num_cores=1, num_subcores=16),
    compiler_params=pltpu.CompilerParams(needs_layout_passes=False),
    scratch_types=[
        pltpu.VMEM((QW,), jnp.int32),
        pltpu.VMEM((QW,), jnp.int32),
        pltpu.VMEM((QW,), jnp.int32),
        pltpu.VMEM((QW // 128, 128), jnp.int32),
        pltpu.VMEM((QW, D), jnp.float32),
        pltpu.VMEM_SHARED((NP,), jnp.int32),
        pltpu.SemaphoreType.DMA,
    ],
)(_sc_sg_body)


def kernel(x, edge_index):
    src = edge_index[0]
    partials = _sc_hist(src)
    keys = _tc_keys(partials, _noise_pad().reshape(NR, 128))
    rank = _tc_rank(keys.reshape(NP, 1), keys.reshape(1, NP))
    sidx_pad, x_sorted = _sc_sg(rank.reshape(NP), x)
    return (x_sorted[None], sidx_pad[:N][None])
